# baseline (speedup 1.0000x reference)
import jax
import jax.numpy as jnp
from jax import lax
from jax.experimental import pallas as pl
from jax.experimental.pallas import tpu as pltpu
import jax.experimental.pallas.tpu_sc as plsc

N = 19
F32 = jnp.float32
I32 = jnp.int32

O_X = 0
O_W1C = 32
O_AS1 = 96
O_AD1 = 160
O_B1 = 224
O_SCAL = 288
O_WCM = 304
O_WM = 336
O_BM = 368
O_AS2 = 400
O_AD2 = 528
O_B2 = 656
O_W2T = 784
O_WCT = 8976
P_LEN = 13072

_NEG = -3.4e38


def _lrelu(v):
    return jnp.maximum(v, 0.2 * v)


def _rsum(vv):
    lane = lax.broadcasted_iota(I32, (16,), 0)
    for sh in (8, 4, 2, 1):
        vv = vv + vv.at[lane ^ sh].get(mode="promise_in_bounds")
    return vv[0]


def _splat(vv, l):
    return vv.at[jnp.full((16,), l, I32)].get(mode="promise_in_bounds")


def _sc_body(p_hbm, out_hbm, P, Hs, H2, OS):
    run = (lax.axis_index("c") == 0) & (lax.axis_index("s") == 0)

    @pl.when(run)
    def _():
        pltpu.sync_copy(p_hbm, P)

        def v(off):
            return P[pl.ds(off, 16)]

        zero = jnp.zeros((16,), F32)

        acc_s = v(O_W1C) * v(O_AS1)
        acc_d = v(O_W1C) * v(O_AD1)
        for j in range(1, 4):
            acc_s = acc_s + v(O_W1C + 16 * j) * v(O_AS1 + 16 * j)
            acc_d = acc_d + v(O_W1C + 16 * j) * v(O_AD1 + 16 * j)
        cs1 = _rsum(acc_s)
        cd1 = _rsum(acc_d)

        x0 = v(O_X)
        x1 = v(O_X + 16)
        xs_l = [x0[l] for l in range(16)] + [x1[l] for l in range(3)]
        ad0 = x0 * cd1
        ad1 = x1 * cd1

        m0 = jnp.full((16,), _NEG, F32)
        m1 = jnp.full((16,), _NEG, F32)
        for s in range(N):
            a = xs_l[s] * cs1
            m0 = jnp.maximum(m0, _lrelu(a + ad0))
            m1 = jnp.maximum(m1, _lrelu(a + ad1))
        den0 = zero
        den1 = zero
        g0 = zero
        g1 = zero
        for s in range(N):
            a = xs_l[s] * cs1
            e0 = jnp.exp(_lrelu(a + ad0) - m0)
            e1 = jnp.exp(_lrelu(a + ad1) - m1)
            den0 = den0 + e0
            den1 = den1 + e1
            g0 = g0 + e0 * xs_l[s]
            g1 = g1 + e1 * xs_l[s]
        gv0 = g0 / den0
        gv1 = g1 / den1
        g_l = [gv0[l] for l in range(16)] + [gv1[l] for l in range(3)]

        w1 = [v(O_W1C + 16 * j) for j in range(4)]
        b1v = [v(O_B1 + 16 * j) for j in range(4)]
        for d in range(N):
            for j in range(4):
                Hs[pl.ds(d * 64 + 16 * j, 16)] = jnp.maximum(
                    g_l[d] * w1[j] + b1v[j], 0.0)

        lane = lax.broadcasted_iota(I32, (16,), 0)
        lane64 = lane * 64
        for blk in range(4):
            s0 = blk * 5
            ns = 5 if blk < 3 else 4

            def body_k(k, carry, s0=s0, ns=ns):
                acc = list(carry)
                wrow = [
                    P[pl.ds(pl.multiple_of(O_W2T + k * 128 + 16 * j, 16), 16)]
                    for j in range(8)
                ]
                hcolA = plsc.load_gather(Hs, [lane64 + k])
                hcolB = (plsc.load_gather(Hs, [lane64 + 64 * 16 + k])
                         if s0 + ns > 16 else None)
                for i in range(ns):
                    s = s0 + i
                    hs = (_splat(hcolA, s) if s < 16 else _splat(hcolB, s - 16))
                    for j in range(8):
                        acc[i * 8 + j] = acc[i * 8 + j] + hs * wrow[j]
                return tuple(acc)

            acc = lax.fori_loop(0, 64, body_k, tuple(zero for _ in range(ns * 8)))
            for i in range(ns):
                for j in range(8):
                    H2[pl.ds((s0 + i) * 128 + 16 * j, 16)] = acc[i * 8 + j]

        b2base = jnp.full((16,), O_B2, I32)
        as2base = jnp.full((16,), O_AS2, I32)
        ad2base = jnp.full((16,), O_AD2, I32)
        d0base = lane * 128
        d1base = (lane + 16) * 128
        SPLIT = 9

        def body_fa(f, carry):
            q = list(carry[:-3])
            pb, as2v0, ad2v0 = carry[-3:]
            w0 = P[pl.ds(pl.multiple_of(O_WCT + f * 32, 16), 16)]
            w1f = P[pl.ds(pl.multiple_of(O_WCT + f * 32 + 16, 16), 16)]
            pb = pb + plsc.load_gather(P, [b2base + f]) * (w0 + w1f)
            hv0 = plsc.load_gather(H2, [d0base + f])
            as2v0 = as2v0 + plsc.load_gather(P, [as2base + f]) * hv0
            ad2v0 = ad2v0 + plsc.load_gather(P, [ad2base + f]) * hv0
            for s in range(SPLIT):
                hf = _splat(hv0, s)
                q[2 * s] = q[2 * s] + hf * w0
                q[2 * s + 1] = q[2 * s + 1] + hf * w1f
            return tuple(q) + (pb, as2v0, ad2v0)

        qa = lax.fori_loop(0, 128, body_fa,
                           tuple(zero for _ in range(2 * SPLIT + 3)), unroll=2)
        pbv, as2v0, ad2v0 = qa[-3:]

        def body_fb(f, carry):
            q = list(carry[:-2])
            as2v1, ad2v1 = carry[-2:]
            w0 = P[pl.ds(pl.multiple_of(O_WCT + f * 32, 16), 16)]
            w1f = P[pl.ds(pl.multiple_of(O_WCT + f * 32 + 16, 16), 16)]
            hv0 = plsc.load_gather(H2, [d0base + f])
            hv1 = plsc.load_gather(H2, [d1base + f])
            as2v1 = as2v1 + plsc.load_gather(P, [as2base + f]) * hv1
            ad2v1 = ad2v1 + plsc.load_gather(P, [ad2base + f]) * hv1
            for s in range(SPLIT, N):
                i = s - SPLIT
                hf = _splat(hv0, s) if s < 16 else _splat(hv1, s - 16)
                q[2 * i] = q[2 * i] + hf * w0
                q[2 * i + 1] = q[2 * i + 1] + hf * w1f
            return tuple(q) + (as2v1, ad2v1)

        qb = lax.fori_loop(0, 128, body_fb,
                           tuple(zero for _ in range(2 * (N - SPLIT) + 2)),
                           unroll=2)
        as2v1, ad2v1 = qb[-2:]
        q = list(qa[:-3]) + list(qb[:-2])

        keep = lane < (N - 16)
        as2v1 = jnp.where(keep, as2v1, 0.0)
        ad2v1 = jnp.where(keep, ad2v1, 0.0)
        as2_l = ([as2v0[l] for l in range(16)]
                 + [as2v1[l] for l in range(N - 16)])

        m20 = jnp.full((16,), _NEG, F32)
        m21 = jnp.full((16,), _NEG, F32)
        for s in range(N):
            m20 = jnp.maximum(m20, _lrelu(as2_l[s] + ad2v0))
            m21 = jnp.maximum(m21, _lrelu(as2_l[s] + ad2v1))
        den20 = zero
        den21 = zero
        for s in range(N):
            den20 = den20 + jnp.exp(_lrelu(as2_l[s] + ad2v0) - m20)
            den21 = den21 + jnp.exp(_lrelu(as2_l[s] + ad2v1) - m21)

        pa0 = zero
        pa1 = zero
        for s in range(N):
            ex0 = jnp.exp(_lrelu(as2_l[s] + ad2v0) - m20)
            ex1 = jnp.exp(_lrelu(as2_l[s] + ad2v1) - m21)
            pa0 = pa0 + ex0 * q[2 * s]
            pa1 = pa1 + ex1 * q[2 * s + 1]
        pa = pa0 / den20 + pa1 / den21
        pred = _rsum(pa)

        pred = pred + _rsum(pbv)

        mm = v(O_SCAL)[0]
        mc0 = mm * v(O_WM) + v(O_BM)
        mc1 = mm * v(O_WM + 16) + v(O_BM + 16)
        t2 = mc0 * v(O_WCM) + mc1 * v(O_WCM + 16)
        pred = pred + _rsum(t2) + v(O_SCAL)[1]

        pv = jnp.broadcast_to(pred, (16,))
        OS[pl.ds(0, 16)] = 1.0 / (1.0 + jnp.exp(-pv))
        pltpu.sync_copy(OS, out_hbm)


def kernel(eeg_dem_scores, mmse, W1, a_src1, a_dst1, b1, W2, a_src2, a_dst2,
           b2, Wm, bm, Wc, bc):
    x = eeg_dem_scores[:, 0]
    wcmat = Wc[0, : N * 128].reshape(N, 128)
    wct = jnp.pad(wcmat.T, ((0, 0), (0, 32 - N)))
    parts = [
        jnp.pad(x, (0, 32 - N)),
        W1[:, 0],
        a_src1, a_dst1, b1,
        jnp.concatenate([mmse, bc, jnp.zeros((14,), F32)]),
        Wc[0, N * 128:],
        Wm[:, 0], bm,
        a_src2, a_dst2, b2,
        W2.T.reshape(-1),
        wct.reshape(-1),
    ]
    p = jnp.concatenate(parts)

    mesh = plsc.VectorSubcoreMesh(core_axis_name="c", subcore_axis_name="s")
    out = pl.kernel(
        _sc_body,
        out_type=jax.ShapeDtypeStruct((16,), F32),
        mesh=mesh,
        compiler_params=pltpu.CompilerParams(needs_layout_passes=False),
        scratch_types=[
            pltpu.VMEM((P_LEN,), F32),
            pltpu.VMEM((32 * 64,), F32),
            pltpu.VMEM((32 * 128,), F32),
            pltpu.VMEM((16,), F32),
        ],
    )(p)
    return out[:1].reshape(1, 1)


# speedup vs baseline: 1.1528x; 1.1528x over previous
import jax
import jax.numpy as jnp
from jax import lax
from jax.experimental import pallas as pl
from jax.experimental.pallas import tpu as pltpu
import jax.experimental.pallas.tpu_sc as plsc

N = 19
F32 = jnp.float32
I32 = jnp.int32

O_X = 0
O_W1C = 32
O_AS1 = 96
O_AD1 = 160
O_B1 = 224
O_SCAL = 288
O_WCM = 304
O_WM = 336
O_BM = 368
O_AS2 = 400
O_AD2 = 528
O_B2 = 656
O_W2T = 784
O_WCT = 8976
P_LEN = 13072

_NEG = -3.4e38


def _lrelu(v):
    return jnp.maximum(v, 0.2 * v)


def _rsum(vv):
    lane = lax.broadcasted_iota(I32, (16,), 0)
    for sh in (8, 4, 2, 1):
        vv = vv + vv.at[lane ^ sh].get(mode="promise_in_bounds")
    return vv[0]


def _splat(vv, l):
    return vv.at[jnp.full((16,), l, I32)].get(mode="promise_in_bounds")


def _sc_body(p_hbm, out_hbm, P, Hs, H2, OS):
    run = (lax.axis_index("c") == 0) & (lax.axis_index("s") == 0)

    @pl.when(run)
    def _():
        pltpu.sync_copy(p_hbm, P)

        def v(off):
            return P[pl.ds(off, 16)]

        zero = jnp.zeros((16,), F32)

        acc_s = v(O_W1C) * v(O_AS1)
        acc_d = v(O_W1C) * v(O_AD1)
        for j in range(1, 4):
            acc_s = acc_s + v(O_W1C + 16 * j) * v(O_AS1 + 16 * j)
            acc_d = acc_d + v(O_W1C + 16 * j) * v(O_AD1 + 16 * j)
        cs1 = _rsum(acc_s)
        cd1 = _rsum(acc_d)

        x0 = v(O_X)
        x1 = v(O_X + 16)
        xs_l = [x0[l] for l in range(16)] + [x1[l] for l in range(3)]
        ad0 = x0 * cd1
        ad1 = x1 * cd1

        m0 = jnp.full((16,), _NEG, F32)
        m1 = jnp.full((16,), _NEG, F32)
        for s in range(N):
            a = xs_l[s] * cs1
            m0 = jnp.maximum(m0, _lrelu(a + ad0))
            m1 = jnp.maximum(m1, _lrelu(a + ad1))
        den0 = zero
        den1 = zero
        g0 = zero
        g1 = zero
        for s in range(N):
            a = xs_l[s] * cs1
            e0 = jnp.exp(_lrelu(a + ad0) - m0)
            e1 = jnp.exp(_lrelu(a + ad1) - m1)
            den0 = den0 + e0
            den1 = den1 + e1
            g0 = g0 + e0 * xs_l[s]
            g1 = g1 + e1 * xs_l[s]
        gv0 = g0 / den0
        gv1 = g1 / den1
        g_l = [gv0[l] for l in range(16)] + [gv1[l] for l in range(3)]

        w1 = [v(O_W1C + 16 * j) for j in range(4)]
        b1v = [v(O_B1 + 16 * j) for j in range(4)]
        for d in range(N):
            for j in range(4):
                Hs[pl.ds(d * 64 + 16 * j, 16)] = jnp.maximum(
                    g_l[d] * w1[j] + b1v[j], 0.0)

        lane = lax.broadcasted_iota(I32, (16,), 0)
        for blk in range(4):
            s0 = blk * 5
            ns = 5 if blk < 3 else 4

            base = [jnp.full((16,), (s0 + i) * 64, I32) for i in range(ns)]

            def body_k(k, carry, base=base, ns=ns):
                acc = list(carry)
                wrow = [
                    P[pl.ds(pl.multiple_of(O_W2T + k * 128 + 16 * j, 16), 16)]
                    for j in range(8)
                ]
                for i in range(ns):
                    hs = plsc.load_gather(Hs, [base[i] + k])
                    for j in range(8):
                        acc[i * 8 + j] = acc[i * 8 + j] + hs * wrow[j]
                return tuple(acc)

            acc = lax.fori_loop(0, 64, body_k, tuple(zero for _ in range(ns * 8)))
            for i in range(ns):
                for j in range(8):
                    H2[pl.ds((s0 + i) * 128 + 16 * j, 16)] = acc[i * 8 + j]

        a2s = [v(O_AS2 + 16 * j) for j in range(8)]
        a2d = [v(O_AD2 + 16 * j) for j in range(8)]
        as2_l = []
        ad2_l = []
        for s in range(N):
            row = [H2[pl.ds(s * 128 + 16 * j, 16)] for j in range(8)]
            ts = row[0] * a2s[0]
            td = row[0] * a2d[0]
            for j in range(1, 8):
                ts = ts + row[j] * a2s[j]
                td = td + row[j] * a2d[j]
            as2_l.append(_rsum(ts))
            ad2_l.append(_rsum(td))
        ad2v0 = zero
        ad2v1 = zero
        for d in range(16):
            ad2v0 = ad2v0 + jnp.where(lane == d, ad2_l[d], 0.0)
        for d in range(16, N):
            ad2v1 = ad2v1 + jnp.where(lane == (d - 16), ad2_l[d], 0.0)

        b2base = jnp.full((16,), O_B2, I32)
        sbase = [jnp.full((16,), s * 128, I32) for s in range(N)]
        SPLIT = 9

        def body_fa(f, carry):
            q = list(carry[:-1])
            pb = carry[-1]
            w0 = P[pl.ds(pl.multiple_of(O_WCT + f * 32, 16), 16)]
            w1f = P[pl.ds(pl.multiple_of(O_WCT + f * 32 + 16, 16), 16)]
            pb = pb + plsc.load_gather(P, [b2base + f]) * (w0 + w1f)
            for s in range(SPLIT):
                hf = plsc.load_gather(H2, [sbase[s] + f])
                q[2 * s] = q[2 * s] + hf * w0
                q[2 * s + 1] = q[2 * s + 1] + hf * w1f
            return tuple(q) + (pb,)

        qa = lax.fori_loop(0, 128, body_fa,
                           tuple(zero for _ in range(2 * SPLIT + 1)), unroll=2)
        pbv = qa[-1]

        def body_fb(f, carry):
            q = list(carry)
            w0 = P[pl.ds(pl.multiple_of(O_WCT + f * 32, 16), 16)]
            w1f = P[pl.ds(pl.multiple_of(O_WCT + f * 32 + 16, 16), 16)]
            for s in range(SPLIT, N):
                i = s - SPLIT
                hf = plsc.load_gather(H2, [sbase[s] + f])
                q[2 * i] = q[2 * i] + hf * w0
                q[2 * i + 1] = q[2 * i + 1] + hf * w1f
            return tuple(q)

        qb = lax.fori_loop(0, 128, body_fb,
                           tuple(zero for _ in range(2 * (N - SPLIT))),
                           unroll=2)
        q = list(qa[:-1]) + list(qb)

        m20 = jnp.full((16,), _NEG, F32)
        m21 = jnp.full((16,), _NEG, F32)
        for s in range(N):
            m20 = jnp.maximum(m20, _lrelu(as2_l[s] + ad2v0))
            m21 = jnp.maximum(m21, _lrelu(as2_l[s] + ad2v1))
        den20 = zero
        den21 = zero
        for s in range(N):
            den20 = den20 + jnp.exp(_lrelu(as2_l[s] + ad2v0) - m20)
            den21 = den21 + jnp.exp(_lrelu(as2_l[s] + ad2v1) - m21)

        pa0 = zero
        pa1 = zero
        for s in range(N):
            ex0 = jnp.exp(_lrelu(as2_l[s] + ad2v0) - m20)
            ex1 = jnp.exp(_lrelu(as2_l[s] + ad2v1) - m21)
            pa0 = pa0 + ex0 * q[2 * s]
            pa1 = pa1 + ex1 * q[2 * s + 1]
        pa = pa0 / den20 + pa1 / den21
        pred = _rsum(pa)

        pred = pred + _rsum(pbv)

        mm = v(O_SCAL)[0]
        mc0 = mm * v(O_WM) + v(O_BM)
        mc1 = mm * v(O_WM + 16) + v(O_BM + 16)
        t2 = mc0 * v(O_WCM) + mc1 * v(O_WCM + 16)
        pred = pred + _rsum(t2) + v(O_SCAL)[1]

        pv = jnp.broadcast_to(pred, (16,))
        OS[pl.ds(0, 16)] = 1.0 / (1.0 + jnp.exp(-pv))
        pltpu.sync_copy(OS, out_hbm)


def kernel(eeg_dem_scores, mmse, W1, a_src1, a_dst1, b1, W2, a_src2, a_dst2,
           b2, Wm, bm, Wc, bc):
    x = eeg_dem_scores[:, 0]
    wcmat = Wc[0, : N * 128].reshape(N, 128)
    wct = jnp.pad(wcmat.T, ((0, 0), (0, 32 - N)))
    parts = [
        jnp.pad(x, (0, 32 - N)),
        W1[:, 0],
        a_src1, a_dst1, b1,
        jnp.concatenate([mmse, bc, jnp.zeros((14,), F32)]),
        Wc[0, N * 128:],
        Wm[:, 0], bm,
        a_src2, a_dst2, b2,
        W2.T.reshape(-1),
        wct.reshape(-1),
    ]
    p = jnp.concatenate(parts)

    mesh = plsc.VectorSubcoreMesh(core_axis_name="c", subcore_axis_name="s")
    out = pl.kernel(
        _sc_body,
        out_type=jax.ShapeDtypeStruct((16,), F32),
        mesh=mesh,
        compiler_params=pltpu.CompilerParams(needs_layout_passes=False),
        scratch_types=[
            pltpu.VMEM((P_LEN,), F32),
            pltpu.VMEM((N * 64,), F32),
            pltpu.VMEM((N * 128,), F32),
            pltpu.VMEM((16,), F32),
        ],
    )(p)
    return out[:1].reshape(1, 1)


# speedup vs baseline: 1.3304x; 1.1540x over previous
import jax
import jax.numpy as jnp
from jax import lax
from jax.experimental import pallas as pl
from jax.experimental.pallas import tpu as pltpu
import jax.experimental.pallas.tpu_sc as plsc

N = 19
F32 = jnp.float32
I32 = jnp.int32

O_X = 0
O_W1C = 32
O_AS1 = 96
O_AD1 = 160
O_B1 = 224
O_SCAL = 288
O_WCM = 304
O_WM = 336
O_BM = 368
O_AS2 = 400
O_AD2 = 528
O_B2 = 656
O_HEAD = 784
O_W2T = 784
O_WCT = 8976
P_LEN = 13072

_NEG = -3.4e38
LSTRIDE = 24
PSTRIDE = 48


def _lrelu(v):
    return jnp.maximum(v, 0.2 * v)


def _rsum(vv):
    lane = lax.broadcasted_iota(I32, (16,), 0)
    for sh in (8, 4, 2, 1):
        vv = vv + vv.at[lane ^ sh].get(mode="promise_in_bounds")
    return vv[0]


def _sc_body(p_hbm, out_hbm, P, Hs, H2, LG, PP, SHL, SHP, sem):
    cid = lax.axis_index("c")
    tid = lax.axis_index("s")
    lane = lax.broadcasted_iota(I32, (16,), 0)
    zero = jnp.zeros((16,), F32)

    pltpu.sync_copy(p_hbm.at[pl.ds(0, O_HEAD)], P.at[pl.ds(0, O_HEAD)])
    big = pltpu.async_copy(p_hbm.at[pl.ds(O_W2T, P_LEN - O_W2T)],
                           P.at[pl.ds(O_W2T, P_LEN - O_W2T)], sem)

    def v(off):
        return P[pl.ds(off, 16)]

    acc_s = v(O_W1C) * v(O_AS1)
    acc_d = v(O_W1C) * v(O_AD1)
    for j in range(1, 4):
        acc_s = acc_s + v(O_W1C + 16 * j) * v(O_AS1 + 16 * j)
        acc_d = acc_d + v(O_W1C + 16 * j) * v(O_AD1 + 16 * j)
    cs1 = _rsum(acc_s)
    cd1 = _rsum(acc_d)

    x0 = v(O_X)
    x1 = v(O_X + 16)
    xs_l = [x0[l] for l in range(16)] + [x1[l] for l in range(3)]
    ad0 = x0 * cd1
    ad1 = x1 * cd1

    m0 = jnp.full((16,), _NEG, F32)
    m1 = jnp.full((16,), _NEG, F32)
    for s in range(N):
        a = xs_l[s] * cs1
        m0 = jnp.maximum(m0, _lrelu(a + ad0))
        m1 = jnp.maximum(m1, _lrelu(a + ad1))
    den0 = zero
    den1 = zero
    g0 = zero
    g1 = zero
    for s in range(N):
        a = xs_l[s] * cs1
        e0 = jnp.exp(_lrelu(a + ad0) - m0)
        e1 = jnp.exp(_lrelu(a + ad1) - m1)
        den0 = den0 + e0
        den1 = den1 + e1
        g0 = g0 + e0 * xs_l[s]
        g1 = g1 + e1 * xs_l[s]
    gv0 = g0 / den0
    gv1 = g1 / den1
    g_l = [gv0[l] for l in range(16)] + [gv1[l] for l in range(3)]

    w1 = [v(O_W1C + 16 * j) for j in range(4)]
    b1v = [v(O_B1 + 16 * j) for j in range(4)]
    for d in range(N):
        for j in range(4):
            Hs[pl.ds(d * 64 + 16 * j, 16)] = jnp.maximum(
                g_l[d] * w1[j] + b1v[j], 0.0)

    big.wait()

    a2s = [v(O_AS2 + 16 * j) for j in range(8)]
    a2d = [v(O_AD2 + 16 * j) for j in range(8)]
    hbase0 = jnp.full((16,), 64, I32) * tid
    hbase1 = hbase0 + 16 * 64

    logit = zero

    def row_work(hbase, h2slot, logit, lpos0):
        def body_k(k, carry):
            acc = list(carry)
            wrow = [
                P[pl.ds(pl.multiple_of(O_W2T + k * 128 + 16 * j, 16), 16)]
                for j in range(8)
            ]
            hs = plsc.load_gather(Hs, [hbase + k])
            for j in range(8):
                acc[j] = acc[j] + hs * wrow[j]
            return tuple(acc)

        acc = lax.fori_loop(0, 64, body_k, tuple(zero for _ in range(8)))
        ts = acc[0] * a2s[0]
        td = acc[0] * a2d[0]
        for j in range(1, 8):
            ts = ts + acc[j] * a2s[j]
            td = td + acc[j] * a2d[j]
        for j in range(8):
            H2[pl.ds(h2slot * 128 + 16 * j, 16)] = acc[j]
        logit = logit + jnp.where(lane == lpos0, _rsum(ts), 0.0)
        logit = logit + jnp.where(lane == lpos0 + 1, _rsum(td), 0.0)
        return logit

    logit = row_work(hbase0, 0, logit, 0)

    @pl.when(tid < N - 16)
    def _():
        LG[pl.ds(0, 16)] = row_work(hbase1, 1, logit, 2)

    @pl.when(tid >= N - 16)
    def _():
        LG[pl.ds(0, 16)] = logit

    pltpu.sync_copy(LG.at[pl.ds(0, 16)],
                    SHL.at[pl.ds(tid * LSTRIDE, 16)])
    plsc.subcore_barrier()

    pltpu.sync_copy(SHL, LG)

    l17 = lane * LSTRIDE
    as2v0 = plsc.load_gather(LG, [l17])
    ad2v0 = plsc.load_gather(LG, [l17 + 1])
    as2v1 = plsc.load_gather(LG, [l17 + 2])
    ad2v1 = plsc.load_gather(LG, [l17 + 3])
    as2_l = ([as2v0[l] for l in range(16)]
             + [as2v1[l] for l in range(N - 16)])

    m20 = jnp.full((16,), _NEG, F32)
    m21 = jnp.full((16,), _NEG, F32)
    for s in range(N):
        m20 = jnp.maximum(m20, _lrelu(as2_l[s] + ad2v0))
        m21 = jnp.maximum(m21, _lrelu(as2_l[s] + ad2v1))
    den20 = zero
    den21 = zero
    for s in range(N):
        den20 = den20 + jnp.exp(_lrelu(as2_l[s] + ad2v0) - m20)
        den21 = den21 + jnp.exp(_lrelu(as2_l[s] + ad2v1) - m21)

    b2base = jnp.full((16,), O_B2, I32)
    sb0 = jnp.full((16,), 0, I32)
    sb1 = jnp.full((16,), 128, I32)
    do_pb = tid == 15
    two_rows = tid < N - 16

    def body_f(f, carry):
        q00, q01, q10, q11, pb = carry
        w0 = P[pl.ds(pl.multiple_of(O_WCT + f * 32, 16), 16)]
        w1f = P[pl.ds(pl.multiple_of(O_WCT + f * 32 + 16, 16), 16)]
        pb = pb + plsc.load_gather(P, [b2base + f]) * (w0 + w1f)
        h0 = plsc.load_gather(H2, [sb0 + f])
        h1 = plsc.load_gather(H2, [sb1 + f])
        q00 = q00 + h0 * w0
        q01 = q01 + h0 * w1f
        q10 = q10 + h1 * w0
        q11 = q11 + h1 * w1f
        return (q00, q01, q10, q11, pb)

    q00, q01, q10, q11, pbv = lax.fori_loop(
        0, 128, body_f, (zero, zero, zero, zero, zero), unroll=2)
    pbv = jnp.where(do_pb, pbv, 0.0)

    def exw(as2_s, qq0, qq1):
        ex0 = jnp.exp(_lrelu(as2_s + ad2v0) - m20)
        ex1 = jnp.exp(_lrelu(as2_s + ad2v1) - m21)
        return ex0 * qq0, ex1 * qq1

    as_own0 = plsc.load_gather(LG, [tid * LSTRIDE + jnp.zeros((16,), I32)])[0]
    pa0, pa1 = exw(as_own0, q00, q01)
    as_own1 = plsc.load_gather(LG, [tid * LSTRIDE + jnp.full((16,), 2, I32)])[0]
    pa0b, pa1b = exw(as_own1, q10, q11)
    pa0 = pa0 + jnp.where(two_rows, pa0b, 0.0)
    pa1 = pa1 + jnp.where(two_rows, pa1b, 0.0)

    PP[pl.ds(0, 16)] = pa0
    PP[pl.ds(16, 16)] = pa1
    PP[pl.ds(32, 16)] = pbv
    pltpu.sync_copy(PP, SHP.at[pl.ds(tid * PSTRIDE, PSTRIDE)])
    plsc.subcore_barrier()

    @pl.when((cid == 0) & (tid == 0))
    def _():
        pltpu.sync_copy(SHP, H2.at[pl.ds(0, 16 * PSTRIDE)])
        sa0 = zero
        sa1 = zero
        spb = zero
        for t in range(16):
            sa0 = sa0 + H2[pl.ds(t * PSTRIDE, 16)]
            sa1 = sa1 + H2[pl.ds(t * PSTRIDE + 16, 16)]
            spb = spb + H2[pl.ds(t * PSTRIDE + 32, 16)]
        pa = sa0 / den20 + sa1 / den21
        pred = _rsum(pa) + _rsum(spb)

        mm = v(O_SCAL)[0]
        mc0 = mm * v(O_WM) + v(O_BM)
        mc1 = mm * v(O_WM + 16) + v(O_BM + 16)
        t2 = mc0 * v(O_WCM) + mc1 * v(O_WCM + 16)
        pred = pred + _rsum(t2) + v(O_SCAL)[1]

        pv = jnp.broadcast_to(pred, (16,))
        LG[pl.ds(0, 16)] = 1.0 / (1.0 + jnp.exp(-pv))
        pltpu.sync_copy(LG.at[pl.ds(0, 16)], out_hbm)


def kernel(eeg_dem_scores, mmse, W1, a_src1, a_dst1, b1, W2, a_src2, a_dst2,
           b2, Wm, bm, Wc, bc):
    x = eeg_dem_scores[:, 0]
    wcmat = Wc[0, : N * 128].reshape(N, 128)
    wct = jnp.pad(wcmat.T, ((0, 0), (0, 32 - N)))
    parts = [
        jnp.pad(x, (0, 32 - N)),
        W1[:, 0],
        a_src1, a_dst1, b1,
        jnp.concatenate([mmse, bc, jnp.zeros((14,), F32)]),
        Wc[0, N * 128:],
        Wm[:, 0], bm,
        a_src2, a_dst2, b2,
        W2.T.reshape(-1),
        wct.reshape(-1),
    ]
    p = jnp.concatenate(parts)

    mesh = plsc.VectorSubcoreMesh(core_axis_name="c", subcore_axis_name="s")
    out = pl.kernel(
        _sc_body,
        out_type=jax.ShapeDtypeStruct((16,), F32),
        mesh=mesh,
        compiler_params=pltpu.CompilerParams(needs_layout_passes=False),
        scratch_types=[
            pltpu.VMEM((P_LEN,), F32),
            pltpu.VMEM((N * 64,), F32),
            pltpu.VMEM((16 * PSTRIDE,), F32),
            pltpu.VMEM((16 * LSTRIDE + 16,), F32),
            pltpu.VMEM((PSTRIDE,), F32),
            pltpu.VMEM_SHARED((16 * LSTRIDE + 16,), F32),
            pltpu.VMEM_SHARED((16 * PSTRIDE,), F32),
            pltpu.SemaphoreType.DMA,
        ],
    )(p)
    return out[:1].reshape(1, 1)


# speedup vs baseline: 1.4755x; 1.1091x over previous
import jax
import jax.numpy as jnp
from jax import lax
from jax.experimental import pallas as pl
from jax.experimental.pallas import tpu as pltpu
import jax.experimental.pallas.tpu_sc as plsc

N = 19
F32 = jnp.float32
I32 = jnp.int32

O_X = 0
O_W1C = 32
O_AS1 = 96
O_AD1 = 160
O_B1 = 224
O_SCAL = 288
O_WCM = 304
O_WM = 336
O_BM = 368
O_AS2 = 400
O_AD2 = 528
O_B2 = 656
O_HEAD = 784
O_W2T = 784
O_WCT = 8976
P_LEN = 13072

_NEG = -3.4e38
LSTRIDE = 24
PSTRIDE = 48


def _lrelu(v):
    return jnp.maximum(v, 0.2 * v)


def _rsum(vv):
    lane = lax.broadcasted_iota(I32, (16,), 0)
    for sh in (8, 4, 2, 1):
        vv = vv + vv.at[lane ^ sh].get(mode="promise_in_bounds")
    return vv[0]


def _sc_body(p_hbm, out_hbm, P, Hs, H2, LG, PP, SHL, SHP, sem):
    cid = lax.axis_index("c")
    tid = lax.axis_index("s")
    lane = lax.broadcasted_iota(I32, (16,), 0)
    zero = jnp.zeros((16,), F32)

    pltpu.sync_copy(p_hbm.at[pl.ds(0, O_HEAD)], P.at[pl.ds(0, O_HEAD)])
    big = pltpu.async_copy(p_hbm.at[pl.ds(O_W2T, P_LEN - O_W2T)],
                           P.at[pl.ds(O_W2T, P_LEN - O_W2T)], sem)

    def v(off):
        return P[pl.ds(off, 16)]

    acc_s = v(O_W1C) * v(O_AS1)
    acc_d = v(O_W1C) * v(O_AD1)
    for j in range(1, 4):
        acc_s = acc_s + v(O_W1C + 16 * j) * v(O_AS1 + 16 * j)
        acc_d = acc_d + v(O_W1C + 16 * j) * v(O_AD1 + 16 * j)
    cs1 = _rsum(acc_s)
    cd1 = _rsum(acc_d)

    x0 = v(O_X)
    x1 = v(O_X + 16)
    xs_l = [x0[l] for l in range(16)] + [x1[l] for l in range(3)]
    ad0 = x0 * cd1
    ad1 = x1 * cd1

    m0 = jnp.full((16,), _NEG, F32)
    m1 = jnp.full((16,), _NEG, F32)
    for s in range(N):
        a = xs_l[s] * cs1
        m0 = jnp.maximum(m0, _lrelu(a + ad0))
        m1 = jnp.maximum(m1, _lrelu(a + ad1))
    den0 = zero
    den1 = zero
    g0 = zero
    g1 = zero
    for s in range(N):
        a = xs_l[s] * cs1
        e0 = jnp.exp(_lrelu(a + ad0) - m0)
        e1 = jnp.exp(_lrelu(a + ad1) - m1)
        den0 = den0 + e0
        den1 = den1 + e1
        g0 = g0 + e0 * xs_l[s]
        g1 = g1 + e1 * xs_l[s]
    gv0 = g0 / den0
    gv1 = g1 / den1
    g_l = [gv0[l] for l in range(16)] + [gv1[l] for l in range(3)]

    w1 = [v(O_W1C + 16 * j) for j in range(4)]
    b1v = [v(O_B1 + 16 * j) for j in range(4)]
    for d in range(N):
        for j in range(4):
            Hs[pl.ds(d * 64 + 16 * j, 16)] = jnp.maximum(
                g_l[d] * w1[j] + b1v[j], 0.0)

    big.wait()

    a2s = [v(O_AS2 + 16 * j) for j in range(8)]
    a2d = [v(O_AD2 + 16 * j) for j in range(8)]
    hbase0 = jnp.full((16,), 64, I32) * tid
    hbase1 = hbase0 + 16 * 64

    logit = zero

    def row_work(hbase, h2slot, logit, lpos0):
        def body_k(k, carry):
            acc = list(carry)
            wrow = [
                P[pl.ds(pl.multiple_of(O_W2T + k * 128 + 16 * j, 16), 16)]
                for j in range(8)
            ]
            hs = plsc.load_gather(Hs, [hbase + k])
            for j in range(8):
                acc[j] = acc[j] + hs * wrow[j]
            return tuple(acc)

        acc = lax.fori_loop(0, 64, body_k, tuple(zero for _ in range(8)))
        ts = acc[0] * a2s[0]
        td = acc[0] * a2d[0]
        for j in range(1, 8):
            ts = ts + acc[j] * a2s[j]
            td = td + acc[j] * a2d[j]
        for j in range(8):
            H2[pl.ds(h2slot * 128 + 16 * j, 16)] = acc[j]
        logit = logit + jnp.where(lane == lpos0, _rsum(ts), 0.0)
        logit = logit + jnp.where(lane == lpos0 + 1, _rsum(td), 0.0)
        return logit

    logit = row_work(hbase0, 0, logit, 0)

    @pl.when(tid < N - 16)
    def _():
        LG[pl.ds(0, 16)] = row_work(hbase1, 1, logit, 2)

    @pl.when(tid >= N - 16)
    def _():
        LG[pl.ds(0, 16)] = logit

    pltpu.sync_copy(LG.at[pl.ds(0, 16)],
                    SHL.at[pl.ds(tid * LSTRIDE, 16)])
    plsc.subcore_barrier()

    pltpu.sync_copy(SHL, LG)

    l17 = lane * LSTRIDE
    as2v0 = plsc.load_gather(LG, [l17])
    ad2v0 = plsc.load_gather(LG, [l17 + 1])
    as2v1 = plsc.load_gather(LG, [l17 + 2])
    ad2v1 = plsc.load_gather(LG, [l17 + 3])
    as2_l = ([as2v0[l] for l in range(16)]
             + [as2v1[l] for l in range(N - 16)])

    m20 = jnp.full((16,), _NEG, F32)
    m21 = jnp.full((16,), _NEG, F32)
    for s in range(N):
        m20 = jnp.maximum(m20, _lrelu(as2_l[s] + ad2v0))
        m21 = jnp.maximum(m21, _lrelu(as2_l[s] + ad2v1))
    den20 = zero
    den21 = zero
    for s in range(N):
        den20 = den20 + jnp.exp(_lrelu(as2_l[s] + ad2v0) - m20)
        den21 = den21 + jnp.exp(_lrelu(as2_l[s] + ad2v1) - m21)

    b2base = jnp.full((16,), O_B2, I32)
    sb0 = jnp.full((16,), 0, I32)
    sb1 = jnp.full((16,), 128, I32)
    do_pb = tid == 15
    two_rows = tid < N - 16

    def body_f(f, carry):
        q00, q01, q10, q11, pb = carry
        w0 = P[pl.ds(pl.multiple_of(O_WCT + f * 32, 16), 16)]
        w1f = P[pl.ds(pl.multiple_of(O_WCT + f * 32 + 16, 16), 16)]
        pb = pb + plsc.load_gather(P, [b2base + f]) * (w0 + w1f)
        h0 = plsc.load_gather(H2, [sb0 + f])
        h1 = plsc.load_gather(H2, [sb1 + f])
        q00 = q00 + h0 * w0
        q01 = q01 + h0 * w1f
        q10 = q10 + h1 * w0
        q11 = q11 + h1 * w1f
        return (q00, q01, q10, q11, pb)

    q00, q01, q10, q11, pbv = lax.fori_loop(
        0, 128, body_f, (zero, zero, zero, zero, zero), unroll=2)
    pbv = jnp.where(do_pb, pbv, 0.0)

    def exw(as2_s, qq0, qq1):
        ex0 = jnp.exp(_lrelu(as2_s + ad2v0) - m20)
        ex1 = jnp.exp(_lrelu(as2_s + ad2v1) - m21)
        return ex0 * qq0, ex1 * qq1

    as_own0 = plsc.load_gather(LG, [tid * LSTRIDE + jnp.zeros((16,), I32)])[0]
    pa0, pa1 = exw(as_own0, q00, q01)
    as_own1 = plsc.load_gather(LG, [tid * LSTRIDE + jnp.full((16,), 2, I32)])[0]
    pa0b, pa1b = exw(as_own1, q10, q11)
    pa0 = pa0 + jnp.where(two_rows, pa0b, 0.0)
    pa1 = pa1 + jnp.where(two_rows, pa1b, 0.0)

    PP[pl.ds(0, 16)] = pa0
    PP[pl.ds(16, 16)] = pa1
    PP[pl.ds(32, 16)] = pbv
    pltpu.sync_copy(PP, SHP.at[pl.ds(tid * PSTRIDE, PSTRIDE)])
    plsc.subcore_barrier()

    @pl.when((cid == 0) & (tid == 0))
    def _():
        pltpu.sync_copy(SHP, H2.at[pl.ds(0, 16 * PSTRIDE)])
        sa0 = zero
        sa1 = zero
        spb = zero
        for t in range(16):
            sa0 = sa0 + H2[pl.ds(t * PSTRIDE, 16)]
            sa1 = sa1 + H2[pl.ds(t * PSTRIDE + 16, 16)]
            spb = spb + H2[pl.ds(t * PSTRIDE + 32, 16)]
        pa = sa0 / den20 + sa1 / den21
        pred = _rsum(pa) + _rsum(spb)

        mm = v(O_SCAL)[0]
        mc0 = mm * v(O_WM) + v(O_BM)
        mc1 = mm * v(O_WM + 16) + v(O_BM + 16)
        t2 = mc0 * v(O_WCM) + mc1 * v(O_WCM + 16)
        pred = pred + _rsum(t2) + v(O_SCAL)[1]

        pv = jnp.broadcast_to(pred, (16,))
        LG[pl.ds(0, 16)] = 1.0 / (1.0 + jnp.exp(-pv))
        pltpu.sync_copy(LG.at[pl.ds(0, 16)], out_hbm)


def kernel(eeg_dem_scores, mmse, W1, a_src1, a_dst1, b1, W2, a_src2, a_dst2,
           b2, Wm, bm, Wc, bc):
    x = eeg_dem_scores[:, 0]
    wcmat = Wc[0, : N * 128].reshape(N, 128)
    wct = jnp.pad(wcmat.T, ((0, 0), (0, 32 - N)))
    parts = [
        jnp.pad(x, (0, 32 - N)),
        W1[:, 0],
        a_src1, a_dst1, b1,
        jnp.concatenate([mmse, bc, jnp.zeros((14,), F32)]),
        Wc[0, N * 128:],
        Wm[:, 0], bm,
        a_src2, a_dst2, b2,
        W2.T.reshape(-1),
        wct.reshape(-1),
    ]
    p = jnp.concatenate(parts)

    mesh = plsc.VectorSubcoreMesh(core_axis_name="c", subcore_axis_name="s",
                                  num_cores=1)
    out = pl.kernel(
        _sc_body,
        out_type=jax.ShapeDtypeStruct((16,), F32),
        mesh=mesh,
        compiler_params=pltpu.CompilerParams(needs_layout_passes=False),
        scratch_types=[
            pltpu.VMEM((P_LEN,), F32),
            pltpu.VMEM((N * 64,), F32),
            pltpu.VMEM((16 * PSTRIDE,), F32),
            pltpu.VMEM((16 * LSTRIDE + 16,), F32),
            pltpu.VMEM((PSTRIDE,), F32),
            pltpu.VMEM_SHARED((16 * LSTRIDE + 16,), F32),
            pltpu.VMEM_SHARED((16 * PSTRIDE,), F32),
            pltpu.SemaphoreType.DMA,
        ],
    )(p)
    return out[:1].reshape(1, 1)
